# baseline (speedup 1.0000x reference)
import functools

import jax
import jax.numpy as jnp
from jax import lax
from jax.experimental import pallas as pl
from jax.experimental.pallas import tpu as pltpu
from jax.experimental.pallas import tpu_sc as plsc

BATCH = 16384
EMBED_DIM = 32
NUM_CORES = 2
NUM_SUBCORES = 16
NUM_WORKERS = NUM_CORES * NUM_SUBCORES
BPW = BATCH // NUM_WORKERS
CHUNK = 128
NCHUNK = BPW // CHUNK
LANES = 16
WB_LEN = 2 * EMBED_DIM * LANES + LANES

_mesh = plsc.VectorSubcoreMesh(
    core_axis_name="c", subcore_axis_name="s", num_cores=NUM_CORES,
    num_subcores=NUM_SUBCORES)


@functools.partial(
    pl.kernel,
    out_type=jax.ShapeDtypeStruct((BATCH,), jnp.float32),
    mesh=_mesh,
    compiler_params=pltpu.CompilerParams(needs_layout_passes=False,
                                         use_tc_tiling_on_sc=False),
    scratch_types=[
        pltpu.VMEM((BPW,), jnp.int32),
        pltpu.VMEM((BPW,), jnp.int32),
        pltpu.VMEM((BPW, EMBED_DIM), jnp.float32),
        pltpu.VMEM((BPW, EMBED_DIM), jnp.float32),
        pltpu.VMEM((WB_LEN,), jnp.float32),
        pltpu.VMEM((BPW,), jnp.float32),
        pltpu.SemaphoreType.DMA,
        pltpu.SemaphoreType.DMA,
    ],
)
def _fused_lookup_head(users_hbm, movies_hbm, ut_hbm, mt_hbm, wb_hbm,
                       out_hbm, uidx, midx, urows, mrows, wb, outv,
                       sem_u, sem_m):
    wid = lax.axis_index("s") * NUM_CORES + lax.axis_index("c")
    base = wid * BPW
    pltpu.sync_copy(users_hbm.at[pl.ds(base, BPW)], uidx)
    pltpu.sync_copy(movies_hbm.at[pl.ds(base, BPW)], midx)
    pltpu.sync_copy(wb_hbm, wb)
    copies = []
    for c in range(NCHUNK):
        sl = pl.ds(c * CHUNK, CHUNK)
        copies.append(
            pltpu.async_copy(ut_hbm.at[uidx.at[sl]], urows.at[sl, :], sem_u))
        copies.append(
            pltpu.async_copy(mt_hbm.at[midx.at[sl]], mrows.at[sl, :], sem_m))
    for cp in copies:
        cp.wait()

    lanes = lax.iota(jnp.int32, LANES)
    bvec = wb[pl.ds(2 * EMBED_DIM * LANES, LANES)]

    def body(g, carry):
        row_ids = g * LANES + lanes
        acc = bvec
        for d in range(EMBED_DIM):
            dd = jnp.full((LANES,), d, jnp.int32)
            wv = wb[pl.ds(d * LANES, LANES)]
            uv = plsc.load_gather(urows, [row_ids, dd])
            acc = acc + uv * wv
        for d in range(EMBED_DIM):
            dd = jnp.full((LANES,), d, jnp.int32)
            wv = wb[pl.ds((EMBED_DIM + d) * LANES, LANES)]
            mv = plsc.load_gather(mrows, [row_ids, dd])
            acc = acc + mv * wv
        outv[pl.ds(g * LANES, LANES)] = acc
        return carry

    lax.fori_loop(0, BPW // LANES, body, 0)
    pltpu.sync_copy(outv, out_hbm.at[pl.ds(base, BPW)])


def kernel(users, movies, user_table, movie_table, W, b):
    wb = jnp.concatenate([jnp.repeat(W.reshape(-1), LANES),
                          jnp.broadcast_to(b.reshape(()), (LANES,))])
    out = _fused_lookup_head(users.astype(jnp.int32), movies.astype(jnp.int32),
                             user_table, movie_table, wb)
    return out.reshape(BATCH, 1)


# speedup vs baseline: 4.2011x; 4.2011x over previous
import functools

import jax
import jax.numpy as jnp
from jax import lax
from jax.experimental import pallas as pl
from jax.experimental.pallas import tpu as pltpu
from jax.experimental.pallas import tpu_sc as plsc

BATCH = 16384
EMBED_DIM = 32
N_USERS = 1000000
N_MOVIES = 100000
NUM_CORES = 2
NUM_SUBCORES = 16
NUM_WORKERS = NUM_CORES * NUM_SUBCORES
BPW = BATCH // NUM_WORKERS
CHUNK = 128
NCHUNK = BPW // CHUNK
LANES = 16


_TC_BLK = 8192


def _colsum_body(t_ref, w_ref, o_ref):
    o_ref[...] = jnp.sum(t_ref[...] * w_ref[...], axis=0)


def _weighted_colsum(table_t, w_col, n):
    grid = (n + _TC_BLK - 1) // _TC_BLK
    return pl.pallas_call(
        _colsum_body,
        grid=(grid,),
        in_specs=[
            pl.BlockSpec((EMBED_DIM, _TC_BLK), lambda i: (0, i)),
            pl.BlockSpec((EMBED_DIM, 1), lambda i: (0, 0)),
        ],
        out_specs=pl.BlockSpec((_TC_BLK,), lambda i: (i,)),
        out_shape=jax.ShapeDtypeStruct((n,), jnp.float32),
    )(table_t, w_col)


_mesh = plsc.VectorSubcoreMesh(
    core_axis_name="c", subcore_axis_name="s", num_cores=NUM_CORES,
    num_subcores=NUM_SUBCORES)


@functools.partial(
    pl.kernel,
    out_type=jax.ShapeDtypeStruct((BATCH,), jnp.float32),
    mesh=_mesh,
    compiler_params=pltpu.CompilerParams(needs_layout_passes=False,
                                         use_tc_tiling_on_sc=False),
    scratch_types=[
        pltpu.VMEM((BPW,), jnp.int32),
        pltpu.VMEM((BPW,), jnp.int32),
        pltpu.VMEM((BPW,), jnp.float32),
        pltpu.VMEM((BPW,), jnp.float32),
        pltpu.VMEM((LANES,), jnp.float32),
        pltpu.VMEM((BPW,), jnp.float32),
        pltpu.SemaphoreType.DMA,
        pltpu.SemaphoreType.DMA,
    ],
)
def _gather_add(users_hbm, movies_hbm, uw_hbm, mw_hbm, b_hbm, out_hbm,
                uidx, midx, gu, gm, bvec, outv, sem_u, sem_m):
    wid = lax.axis_index("s") * NUM_CORES + lax.axis_index("c")
    base = wid * BPW
    pltpu.sync_copy(users_hbm.at[pl.ds(base, BPW)], uidx)
    pltpu.sync_copy(movies_hbm.at[pl.ds(base, BPW)], midx)
    pltpu.sync_copy(b_hbm, bvec)
    copies = []
    for c in range(NCHUNK):
        sl = pl.ds(c * CHUNK, CHUNK)
        copies.append(pltpu.async_copy(uw_hbm.at[uidx.at[sl]], gu.at[sl],
                                       sem_u))
        copies.append(pltpu.async_copy(mw_hbm.at[midx.at[sl]], gm.at[sl],
                                       sem_m))
    for cp in copies:
        cp.wait()
    b_val = bvec[...]
    for s in range(BPW // LANES):
        sl = pl.ds(s * LANES, LANES)
        outv[sl] = gu[sl] + gm[sl] + b_val
    pltpu.sync_copy(outv, out_hbm.at[pl.ds(base, BPW)])


def kernel(users, movies, user_table, movie_table, W, b):
    w = W.reshape(-1)
    uw = _weighted_colsum(user_table.T, w[:EMBED_DIM].reshape(EMBED_DIM, 1),
                          N_USERS)
    mw = _weighted_colsum(movie_table.T, w[EMBED_DIM:].reshape(EMBED_DIM, 1),
                          N_MOVIES)
    bvec = jnp.broadcast_to(b.reshape(()), (LANES,))
    out = _gather_add(users.astype(jnp.int32), movies.astype(jnp.int32),
                      uw, mw, bvec)
    return out.reshape(BATCH, 1)


# speedup vs baseline: 7.4396x; 1.7708x over previous
import functools

import jax
import jax.numpy as jnp
from jax import lax
from jax.experimental import pallas as pl
from jax.experimental.pallas import tpu as pltpu
from jax.experimental.pallas import tpu_sc as plsc

BATCH = 16384
EMBED_DIM = 32
N_USERS = 1000000
N_MOVIES = 100000
NUM_CORES = 2
NUM_SUBCORES = 16
NUM_WORKERS = NUM_CORES * NUM_SUBCORES
BPW = BATCH // NUM_WORKERS
CHUNK = 128
NCHUNK = BPW // CHUNK
LANES = 16


_TC_BLK = 65536


def _colsum_body(t_ref, w_ref, o_ref):
    o_ref[...] = jnp.sum(t_ref[...] * w_ref[...], axis=0)


def _weighted_colsum(table_t, w_col, n):
    grid = (n + _TC_BLK - 1) // _TC_BLK
    return pl.pallas_call(
        _colsum_body,
        grid=(grid,),
        in_specs=[
            pl.BlockSpec((EMBED_DIM, _TC_BLK), lambda i: (0, i)),
            pl.BlockSpec((EMBED_DIM, 1), lambda i: (0, 0)),
        ],
        out_specs=pl.BlockSpec((_TC_BLK,), lambda i: (i,)),
        out_shape=jax.ShapeDtypeStruct((n,), jnp.float32),
    )(table_t, w_col)


_mesh = plsc.VectorSubcoreMesh(
    core_axis_name="c", subcore_axis_name="s", num_cores=NUM_CORES,
    num_subcores=NUM_SUBCORES)


@functools.partial(
    pl.kernel,
    out_type=jax.ShapeDtypeStruct((BATCH,), jnp.float32),
    mesh=_mesh,
    compiler_params=pltpu.CompilerParams(needs_layout_passes=False,
                                         use_tc_tiling_on_sc=False),
    scratch_types=[
        pltpu.VMEM((BPW,), jnp.int32),
        pltpu.VMEM((BPW,), jnp.int32),
        pltpu.VMEM((BPW,), jnp.float32),
        pltpu.VMEM((BPW,), jnp.float32),
        pltpu.VMEM((LANES,), jnp.float32),
        pltpu.VMEM((BPW,), jnp.float32),
        pltpu.SemaphoreType.DMA,
        pltpu.SemaphoreType.DMA,
    ],
)
def _gather_add(users_hbm, movies_hbm, uw_hbm, mw_hbm, b_hbm, out_hbm,
                uidx, midx, gu, gm, bvec, outv, sem_u, sem_m):
    wid = lax.axis_index("s") * NUM_CORES + lax.axis_index("c")
    base = wid * BPW
    pltpu.sync_copy(users_hbm.at[pl.ds(base, BPW)], uidx)
    pltpu.sync_copy(movies_hbm.at[pl.ds(base, BPW)], midx)
    pltpu.sync_copy(b_hbm, bvec)
    copies = []
    for c in range(NCHUNK):
        sl = pl.ds(c * CHUNK, CHUNK)
        copies.append(pltpu.async_copy(uw_hbm.at[uidx.at[sl]], gu.at[sl],
                                       sem_u))
        copies.append(pltpu.async_copy(mw_hbm.at[midx.at[sl]], gm.at[sl],
                                       sem_m))
    for cp in copies:
        cp.wait()
    b_val = bvec[...]
    for s in range(BPW // LANES):
        sl = pl.ds(s * LANES, LANES)
        outv[sl] = gu[sl] + gm[sl] + b_val
    pltpu.sync_copy(outv, out_hbm.at[pl.ds(base, BPW)])


def kernel(users, movies, user_table, movie_table, W, b):
    w = W.reshape(-1)
    uw = _weighted_colsum(user_table.T, w[:EMBED_DIM].reshape(EMBED_DIM, 1),
                          N_USERS)
    mw = _weighted_colsum(movie_table.T, w[EMBED_DIM:].reshape(EMBED_DIM, 1),
                          N_MOVIES)
    bvec = jnp.broadcast_to(b.reshape(()), (LANES,))
    out = _gather_add(users.astype(jnp.int32), movies.astype(jnp.int32),
                      uw, mw, bvec)
    return out.reshape(BATCH, 1)
